# Initial kernel scaffold; baseline (speedup 1.0000x reference)
import jax
import jax.numpy as jnp
from jax.experimental import pallas as pl


def kernel(contributions):
    raise NotImplementedError("write your pallas kernel here")


# speedup vs baseline: 8.3149x; 8.3149x over previous
import functools

import jax
import jax.numpy as jnp
from jax import lax
from jax.experimental import pallas as pl
from jax.experimental.pallas import tpu as pltpu
from jax.experimental.pallas import tpu_sc as plsc

N_SAMPLES, N_PROTO, N_CLASS = 64, 2048, 256
L = 16
NC, NS = 2, 16
NW = NC * NS
CB = N_CLASS // L
N_ITEMS = N_SAMPLES * CB
ITEMS_PER_W = N_ITEMS // NW


def _tec_body(x_hbm, out_hbm, buf):
    wid = lax.axis_index("s") * NC + lax.axis_index("c")

    def item_body(j, carry):
        item = wid * ITEMS_PER_W + j
        s = item // CB
        col = (item % CB) * L
        pltpu.sync_copy(x_hbm.at[s, :, pl.ds(col, L)], buf)

        neg_inf = jnp.full((L,), -jnp.inf, jnp.float32)

        def scan_body(p, tops):
            t1, t2, t3, t4 = tops
            v = buf[p, :]
            m1 = jnp.maximum(t1, v)
            v1 = jnp.minimum(t1, v)
            m2 = jnp.maximum(t2, v1)
            v2 = jnp.minimum(t2, v1)
            m3 = jnp.maximum(t3, v2)
            v3 = jnp.minimum(t3, v2)
            m4 = jnp.maximum(t4, v3)
            return (m1, m2, m3, m4)

        tops = lax.fori_loop(0, N_PROTO, scan_body, (neg_inf,) * 4)
        t4 = tops[3]

        def mask_body(p, _):
            v = buf[p, :]
            buf[p, :] = jnp.where(v >= t4, 0.0, v)
            return 0

        lax.fori_loop(0, N_PROTO, mask_body, 0)
        pltpu.sync_copy(buf, out_hbm.at[s, :, pl.ds(col, L)])
        return carry

    lax.fori_loop(0, ITEMS_PER_W, item_body, 0)


_mesh = plsc.VectorSubcoreMesh(core_axis_name="c", subcore_axis_name="s")

_sc_call = functools.partial(
    pl.kernel,
    mesh=_mesh,
    out_type=jax.ShapeDtypeStruct((N_SAMPLES, N_PROTO, N_CLASS), jnp.float32),
    scratch_types=[pltpu.VMEM((N_PROTO, L), jnp.float32)],
    compiler_params=pltpu.CompilerParams(use_tc_tiling_on_sc=False),
)(_tec_body)


def kernel(contributions):
    return _sc_call(contributions)


# speedup vs baseline: 14.7141x; 1.7696x over previous
import functools

import jax
import jax.numpy as jnp
from jax import lax
from jax.experimental import pallas as pl
from jax.experimental.pallas import tpu as pltpu
from jax.experimental.pallas import tpu_sc as plsc

N_SAMPLES, N_PROTO, N_CLASS = 64, 2048, 256
L = 16
NC, NS = 2, 16
NW = NC * NS
CB = N_CLASS // L
N_ITEMS = N_SAMPLES * CB
ITEMS_PER_W = N_ITEMS // NW


UNR = 8
CHAINS = 4


def _insert(tops, v):
    t1, t2, t3, t4 = tops
    m1 = jnp.maximum(t1, v)
    v1 = jnp.minimum(t1, v)
    m2 = jnp.maximum(t2, v1)
    v2 = jnp.minimum(t2, v1)
    m3 = jnp.maximum(t3, v2)
    v3 = jnp.minimum(t3, v2)
    m4 = jnp.maximum(t4, v3)
    return (m1, m2, m3, m4)


def _tec_body(x_hbm, out_hbm, buf):
    wid = lax.axis_index("s") * NC + lax.axis_index("c")

    def item_body(j, carry):
        item = wid * ITEMS_PER_W + j
        s = item // CB
        col = (item % CB) * L
        pltpu.sync_copy(x_hbm.at[s, :, pl.ds(col, L)], buf)

        neg_inf = jnp.full((L,), -jnp.inf, jnp.float32)
        init = (neg_inf,) * (4 * CHAINS)

        @plsc.parallel_loop(0, N_PROTO, UNR, carry=init)
        def chains(p, chains):
            chains = list(chains)
            for k in range(UNR):
                c = k % CHAINS
                chains[4 * c:4 * c + 4] = _insert(
                    chains[4 * c:4 * c + 4], buf[p + k, :])
            return tuple(chains)

        tops = tuple(chains[:4])
        for c in range(1, CHAINS):
            for tv in chains[4 * c:4 * c + 4]:
                tops = _insert(tops, tv)
        t4 = tops[3]

        @plsc.parallel_loop(0, N_PROTO, UNR, unroll=2)
        def _(p):
            for k in range(UNR):
                v = buf[p + k, :]
                buf[p + k, :] = jnp.where(v >= t4, 0.0, v)

        pltpu.sync_copy(buf, out_hbm.at[s, :, pl.ds(col, L)])
        return carry

    lax.fori_loop(0, ITEMS_PER_W, item_body, 0)


_mesh = plsc.VectorSubcoreMesh(core_axis_name="c", subcore_axis_name="s")

_sc_call = functools.partial(
    pl.kernel,
    mesh=_mesh,
    out_type=jax.ShapeDtypeStruct((N_SAMPLES, N_PROTO, N_CLASS), jnp.float32),
    scratch_types=[pltpu.VMEM((N_PROTO, L), jnp.float32)],
    compiler_params=pltpu.CompilerParams(use_tc_tiling_on_sc=False),
)(_tec_body)


def kernel(contributions):
    return _sc_call(contributions)


# speedup vs baseline: 20.9057x; 1.4208x over previous
import functools

import jax
import jax.numpy as jnp
from jax import lax
from jax.experimental import pallas as pl
from jax.experimental.pallas import tpu as pltpu
from jax.experimental.pallas import tpu_sc as plsc

N_SAMPLES, N_PROTO, N_CLASS = 64, 2048, 256
L = 16
NC, NS = 2, 16
NW = NC * NS
CB = N_CLASS // L
N_ITEMS = N_SAMPLES * CB
ITEMS_PER_W = N_ITEMS // NW
NBUF = 3

UNR = 8
CHAINS = 4


def _insert(tops, v):
    t1, t2, t3, t4 = tops
    m1 = jnp.maximum(t1, v)
    v1 = jnp.minimum(t1, v)
    m2 = jnp.maximum(t2, v1)
    v2 = jnp.minimum(t2, v1)
    m3 = jnp.maximum(t3, v2)
    v3 = jnp.minimum(t3, v2)
    m4 = jnp.maximum(t4, v3)
    return (m1, m2, m3, m4)


def _compute(buf):
    neg_inf = jnp.full((L,), -jnp.inf, jnp.float32)
    init = (neg_inf,) * (4 * CHAINS)

    @plsc.parallel_loop(0, N_PROTO, UNR, carry=init)
    def chains(p, chains):
        chains = list(chains)
        for k in range(UNR):
            c = k % CHAINS
            chains[4 * c:4 * c + 4] = _insert(
                chains[4 * c:4 * c + 4], buf[p + k, :])
        return tuple(chains)

    tops = tuple(chains[:4])
    for c in range(1, CHAINS):
        for tv in chains[4 * c:4 * c + 4]:
            tops = _insert(tops, tv)
    t4 = tops[3]

    @plsc.parallel_loop(0, N_PROTO, UNR, unroll=2)
    def _(p):
        for k in range(UNR):
            v = buf[p + k, :]
            buf[p + k, :] = jnp.where(v >= t4, 0.0, v)


def _tec_body(x_hbm, out_hbm, bufs, in_sems, out_sems):
    wid = lax.axis_index("s") * NC + lax.axis_index("c")

    def src(i):
        item = i * NW + wid
        s = item // CB
        col = (item % CB) * L
        return x_hbm.at[s, :, pl.ds(col, L)]

    def dst(i):
        item = i * NW + wid
        s = item // CB
        col = (item % CB) * L
        return out_hbm.at[s, :, pl.ds(col, L)]

    def stage(i, b, first, last):
        pltpu.make_async_copy(src(i), bufs.at[b], in_sems.at[b]).wait()
        _compute(bufs.at[b])
        pltpu.async_copy(bufs.at[b], dst(i), out_sems.at[b])
        nb = (b + 2) % NBUF
        if not last:
            if not first:
                pltpu.make_async_copy(
                    bufs.at[nb], dst(i - 1), out_sems.at[nb]).wait()
            pltpu.async_copy(src(i + 2), bufs.at[nb], in_sems.at[nb])

    pltpu.async_copy(src(0), bufs.at[0], in_sems.at[0])
    pltpu.async_copy(src(1), bufs.at[1], in_sems.at[1])
    stage(0, 0, True, False)
    stage(1, 1, False, False)
    stage(2, 2, False, False)

    def macro(m, carry):
        i = 3 * m
        stage(i, 0, False, False)
        stage(i + 1, 1, False, False)
        stage(i + 2, 2, False, False)
        return carry

    lax.fori_loop(1, ITEMS_PER_W // 3, macro, 0)

    stage(ITEMS_PER_W - 2, 0, False, True)
    stage(ITEMS_PER_W - 1, 1, False, True)
    for b in range(NBUF):
        i = ITEMS_PER_W - 3 + b
        pltpu.make_async_copy(bufs.at[b], dst(i), out_sems.at[b]).wait()


_mesh = plsc.VectorSubcoreMesh(core_axis_name="c", subcore_axis_name="s")

_sc_call = functools.partial(
    pl.kernel,
    mesh=_mesh,
    out_type=jax.ShapeDtypeStruct((N_SAMPLES, N_PROTO, N_CLASS), jnp.float32),
    scratch_types=[
        pltpu.VMEM((NBUF, N_PROTO, L), jnp.float32),
        pltpu.SemaphoreType.DMA((NBUF,)),
        pltpu.SemaphoreType.DMA((NBUF,)),
    ],
    compiler_params=pltpu.CompilerParams(use_tc_tiling_on_sc=False),
)(_tec_body)


def kernel(contributions):
    return _sc_call(contributions)


# speedup vs baseline: 37.2040x; 1.7796x over previous
import functools

import jax
import jax.numpy as jnp
from jax import lax
from jax.experimental import pallas as pl
from jax.experimental.pallas import tpu as pltpu
from jax.experimental.pallas import tpu_sc as plsc

N_SAMPLES, N_PROTO, N_CLASS = 64, 2048, 256
L = 16
NC, NS = 2, 16
NW = NC * NS
CBLK = 128
N_CBLK = N_CLASS // CBLK
G = CBLK // L
N_ITEMS = N_SAMPLES * N_CBLK
ITEMS_PER_W = N_ITEMS // NW
CHUNK = 128
NCHUNK = N_PROTO // CHUNK
NBUF = 4


def _insert(tops, v):
    t1, t2, t3, t4 = tops
    m1 = jnp.maximum(t1, v)
    v1 = jnp.minimum(t1, v)
    m2 = jnp.maximum(t2, v1)
    v2 = jnp.minimum(t2, v1)
    m3 = jnp.maximum(t3, v2)
    v3 = jnp.minimum(t3, v2)
    m4 = jnp.maximum(t4, v3)
    return (m1, m2, m3, m4)


def _tec_body(x_hbm, out_hbm, bufs, in_sems, out_sems):
    wid = lax.axis_index("s") * NC + lax.axis_index("c")
    neg_inf = jnp.full((L,), -jnp.inf, jnp.float32)

    def item_body(j, carry):
        item = j * NW + wid
        s = item // N_CBLK
        cb = (item % N_CBLK) * CBLK

        def src(c):
            return x_hbm.at[s, pl.ds(c * CHUNK, CHUNK), pl.ds(cb, CBLK)]

        def dst(c):
            return out_hbm.at[s, pl.ds(c * CHUNK, CHUNK), pl.ds(cb, CBLK)]

        for c in range(NBUF - 1):
            pltpu.async_copy(src(c), bufs.at[c], in_sems.at[c])

        glob = [neg_inf] * (4 * G)
        for c in range(NCHUNK):
            b = c % NBUF
            pltpu.make_async_copy(src(c), bufs.at[b], in_sems.at[b]).wait()
            if c + NBUF - 1 < NCHUNK:
                nb = (c + NBUF - 1) % NBUF
                pltpu.async_copy(src(c + NBUF - 1), bufs.at[nb],
                                 in_sems.at[nb])
            buf = bufs.at[b]
            for half in range(2):
                init = tuple(glob[4 * half * 4:4 * half * 4 + 16])

                @plsc.parallel_loop(0, CHUNK, 1, unroll=2, carry=init)
                def sweep(p, loc):
                    loc = list(loc)
                    for g in range(4):
                        gg = half * 4 + g
                        v = buf[p, pl.ds(L * gg, L)]
                        loc[4 * g:4 * g + 4] = _insert(
                            loc[4 * g:4 * g + 4], v)
                    return tuple(loc)

                glob[4 * half * 4:4 * half * 4 + 16] = list(sweep)

        t4s = [glob[4 * g + 3] for g in range(G)]

        pltpu.async_copy(src(0), bufs.at[0], in_sems.at[0])
        pltpu.async_copy(src(1), bufs.at[1], in_sems.at[1])
        for c in range(NCHUNK):
            b = c % NBUF
            pltpu.make_async_copy(src(c), bufs.at[b], in_sems.at[b]).wait()
            buf = bufs.at[b]

            @plsc.parallel_loop(0, CHUNK, 1, unroll=2)
            def _(p):
                for g in range(G):
                    v = buf[p, pl.ds(L * g, L)]
                    buf[p, pl.ds(L * g, L)] = jnp.where(
                        v >= t4s[g], 0.0, v)

            pltpu.async_copy(buf, dst(c), out_sems.at[b])
            if c + 2 < NCHUNK:
                nb = (c + 2) % NBUF
                if c >= 2:
                    pltpu.make_async_copy(bufs.at[nb], dst(c - 2),
                                          out_sems.at[nb]).wait()
                pltpu.async_copy(src(c + 2), bufs.at[nb], in_sems.at[nb])

        for c in range(NCHUNK - NBUF, NCHUNK):
            b = c % NBUF
            pltpu.make_async_copy(bufs.at[b], dst(c), out_sems.at[b]).wait()
        return carry

    lax.fori_loop(0, ITEMS_PER_W, item_body, 0)


_mesh = plsc.VectorSubcoreMesh(core_axis_name="c", subcore_axis_name="s")

_sc_call = functools.partial(
    pl.kernel,
    mesh=_mesh,
    out_type=jax.ShapeDtypeStruct((N_SAMPLES, N_PROTO, N_CLASS), jnp.float32),
    scratch_types=[
        pltpu.VMEM((NBUF, CHUNK, CBLK), jnp.float32),
        pltpu.SemaphoreType.DMA((NBUF,)),
        pltpu.SemaphoreType.DMA((NBUF,)),
    ],
)(_tec_body)


def kernel(contributions):
    return _sc_call(contributions)


# speedup vs baseline: 38.3314x; 1.0303x over previous
import functools

import jax
import jax.numpy as jnp
from jax import lax
from jax.experimental import pallas as pl
from jax.experimental.pallas import tpu as pltpu
from jax.experimental.pallas import tpu_sc as plsc

N_SAMPLES, N_PROTO, N_CLASS = 64, 2048, 256
L = 16
NC, NS = 2, 16
NW = NC * NS
CBLK = 128
N_CBLK = N_CLASS // CBLK
G = CBLK // L
N_ITEMS = N_SAMPLES * N_CBLK
ITEMS_PER_W = N_ITEMS // NW
CHUNK = 128
NCHUNK = N_PROTO // CHUNK
NBUF = 6
P1_AHEAD = 3
P2_AHEAD = 4


def _insert(tops, v):
    t1, t2, t3, t4 = tops
    m1 = jnp.maximum(t1, v)
    v1 = jnp.minimum(t1, v)
    m2 = jnp.maximum(t2, v1)
    v2 = jnp.minimum(t2, v1)
    m3 = jnp.maximum(t3, v2)
    v3 = jnp.minimum(t3, v2)
    m4 = jnp.maximum(t4, v3)
    return (m1, m2, m3, m4)


def _tec_body(x_hbm, out_hbm, bufs, in_sems, out_sems):
    wid = lax.axis_index("s") * NC + lax.axis_index("c")
    neg_inf = jnp.full((L,), -jnp.inf, jnp.float32)

    def item_body(j, carry):
        item = j * NW + wid
        s = item // N_CBLK
        cb = (item % N_CBLK) * CBLK

        def src(c):
            return x_hbm.at[s, pl.ds(c * CHUNK, CHUNK), pl.ds(cb, CBLK)]

        def dst(c):
            return out_hbm.at[s, pl.ds(c * CHUNK, CHUNK), pl.ds(cb, CBLK)]

        for c in range(P1_AHEAD):
            pltpu.async_copy(src(c), bufs.at[c], in_sems.at[c])

        glob = [neg_inf] * (4 * G)
        for c in range(NCHUNK):
            b = c % NBUF
            pltpu.make_async_copy(src(c), bufs.at[b], in_sems.at[b]).wait()
            if c + P1_AHEAD < NCHUNK:
                nb = (c + P1_AHEAD) % NBUF
                pltpu.async_copy(src(c + P1_AHEAD), bufs.at[nb],
                                 in_sems.at[nb])
            buf = bufs.at[b]
            for half in range(2):
                init = tuple(glob[4 * half * 4:4 * half * 4 + 16])

                def sweep_body(i, loc, half=half, buf=buf):
                    loc = list(loc)
                    for k in range(2):
                        p = 2 * i + k
                        for g in range(4):
                            gg = half * 4 + g
                            v = buf[p, pl.ds(L * gg, L)]
                            loc[4 * g:4 * g + 4] = _insert(
                                loc[4 * g:4 * g + 4], v)
                    return tuple(loc)

                fin = lax.fori_loop(0, CHUNK // 2, sweep_body, init)
                glob[4 * half * 4:4 * half * 4 + 16] = list(fin)

        t4s = [glob[4 * g + 3] for g in range(G)]

        for c in range(P2_AHEAD):
            pltpu.async_copy(src(c), bufs.at[c], in_sems.at[c])
        for c in range(NCHUNK):
            b = c % NBUF
            pltpu.make_async_copy(src(c), bufs.at[b], in_sems.at[b]).wait()
            buf = bufs.at[b]

            def mask_body(i, carry, buf=buf):
                for k in range(4):
                    p = 4 * i + k
                    for g in range(G):
                        v = buf[p, pl.ds(L * g, L)]
                        buf[p, pl.ds(L * g, L)] = jnp.where(
                            v >= t4s[g], 0.0, v)
                return carry

            lax.fori_loop(0, CHUNK // 4, mask_body, 0)

            pltpu.async_copy(buf, dst(c), out_sems.at[b])
            if c + P2_AHEAD < NCHUNK:
                nb = (c + P2_AHEAD) % NBUF
                prev = c + P2_AHEAD - NBUF
                if prev >= 0:
                    pltpu.make_async_copy(bufs.at[nb], dst(prev),
                                          out_sems.at[nb]).wait()
                pltpu.async_copy(src(c + P2_AHEAD), bufs.at[nb],
                                 in_sems.at[nb])

        for c in range(NCHUNK - NBUF, NCHUNK):
            b = c % NBUF
            pltpu.make_async_copy(bufs.at[b], dst(c), out_sems.at[b]).wait()
        return carry

    lax.fori_loop(0, ITEMS_PER_W, item_body, 0)


_mesh = plsc.VectorSubcoreMesh(core_axis_name="c", subcore_axis_name="s")

_sc_call = functools.partial(
    pl.kernel,
    mesh=_mesh,
    out_type=jax.ShapeDtypeStruct((N_SAMPLES, N_PROTO, N_CLASS), jnp.float32),
    scratch_types=[
        pltpu.VMEM((NBUF, CHUNK, CBLK), jnp.float32),
        pltpu.SemaphoreType.DMA((NBUF,)),
        pltpu.SemaphoreType.DMA((NBUF,)),
    ],
)(_tec_body)


def kernel(contributions):
    return _sc_call(contributions)


# speedup vs baseline: 41.0379x; 1.0706x over previous
import functools

import jax
import jax.numpy as jnp
from jax import lax
from jax.experimental import pallas as pl
from jax.experimental.pallas import tpu as pltpu
from jax.experimental.pallas import tpu_sc as plsc

N_SAMPLES, N_PROTO, N_CLASS = 64, 2048, 256
L = 16
NC, NS = 2, 16
NW = NC * NS
CBLK = 128
N_CBLK = N_CLASS // CBLK
G = CBLK // L
N_ITEMS = N_SAMPLES * N_CBLK
ITEMS_PER_W = N_ITEMS // NW
CHUNK = 128
NCHUNK = N_PROTO // CHUNK
NBUF = 6
P1_AHEAD = 3
P2_AHEAD = 4


def _insert(tops, v):
    t1, t2, t3, t4 = tops
    m1 = jnp.maximum(t1, v)
    v1 = jnp.minimum(t1, v)
    m2 = jnp.maximum(t2, v1)
    v2 = jnp.minimum(t2, v1)
    m3 = jnp.maximum(t3, v2)
    v3 = jnp.minimum(t3, v2)
    m4 = jnp.maximum(t4, v3)
    return (m1, m2, m3, m4)


def _tec_body(x_hbm, out_hbm, bufs, in_sems, out_sems):
    wid = lax.axis_index("s") * NC + lax.axis_index("c")
    neg_inf = jnp.full((L,), -jnp.inf, jnp.float32)

    def item_body(j, carry):
        item = j * NW + wid
        s = item // N_CBLK
        cb = (item % N_CBLK) * CBLK

        def src(c):
            return x_hbm.at[s, pl.ds(c * CHUNK, CHUNK), pl.ds(cb, CBLK)]

        def dst(c):
            return out_hbm.at[s, pl.ds(c * CHUNK, CHUNK), pl.ds(cb, CBLK)]

        for c in range(P1_AHEAD):
            pltpu.async_copy(src(c), bufs.at[c], in_sems.at[c])

        glob = [neg_inf] * (4 * G)
        for c in range(NCHUNK):
            b = c % NBUF
            pltpu.make_async_copy(src(c), bufs.at[b], in_sems.at[b]).wait()
            if c + P1_AHEAD < NCHUNK:
                nb = (c + P1_AHEAD) % NBUF
                pltpu.async_copy(src(c + P1_AHEAD), bufs.at[nb],
                                 in_sems.at[nb])
            buf = bufs.at[b]
            for half in range(2):
                init = tuple(glob[4 * half * 4:4 * half * 4 + 16])

                def sweep_body(i, loc, half=half, buf=buf):
                    loc = list(loc)
                    for k in range(2):
                        p = 2 * i + k
                        for g in range(4):
                            gg = half * 4 + g
                            v = buf[p, pl.ds(L * gg, L)]
                            loc[4 * g:4 * g + 4] = _insert(
                                loc[4 * g:4 * g + 4], v)
                    return tuple(loc)

                fin = lax.fori_loop(0, CHUNK // 2, sweep_body, init)
                glob[4 * half * 4:4 * half * 4 + 16] = list(fin)

        t4s = [glob[4 * g + 3] for g in range(G)]

        def mask_chunk(b):
            buf = bufs.at[b]

            def mask_body(i, carry, buf=buf):
                for k in range(4):
                    p = 4 * i + k
                    for g in range(G):
                        v = buf[p, pl.ds(L * g, L)]
                        buf[p, pl.ds(L * g, L)] = jnp.where(
                            v >= t4s[g], 0.0, v)
                return carry

            lax.fori_loop(0, CHUNK // 4, mask_body, 0)

        RES0 = NCHUNK - NBUF
        NSTR = NCHUNK - NBUF

        def sbuf(c):
            return (RES0 % NBUF + c) % NBUF

        for k in range(NBUF):
            b = (RES0 + k) % NBUF
            mask_chunk(b)
            pltpu.async_copy(bufs.at[b], dst(RES0 + k), out_sems.at[b])
            if k >= 2:
                j = k - 2
                bj = (RES0 + j) % NBUF
                pltpu.make_async_copy(bufs.at[bj], dst(RES0 + j),
                                      out_sems.at[bj]).wait()
                pltpu.async_copy(src(j), bufs.at[bj], in_sems.at[bj])
        for j in range(NBUF - 2, NBUF):
            bj = (RES0 + j) % NBUF
            pltpu.make_async_copy(bufs.at[bj], dst(RES0 + j),
                                  out_sems.at[bj]).wait()
            pltpu.async_copy(src(j), bufs.at[bj], in_sems.at[bj])

        for c in range(NSTR):
            b = sbuf(c)
            pltpu.make_async_copy(src(c), bufs.at[b], in_sems.at[b]).wait()
            mask_chunk(b)
            pltpu.async_copy(bufs.at[b], dst(c), out_sems.at[b])
            nxt = c + 2
            if c >= 4 and NBUF <= nxt < NSTR:
                pltpu.make_async_copy(bufs.at[sbuf(c - 4)], dst(c - 4),
                                      out_sems.at[sbuf(c - 4)]).wait()
                pltpu.async_copy(src(nxt), bufs.at[sbuf(nxt)],
                                 in_sems.at[sbuf(nxt)])

        for c in range(NSTR - NBUF, NSTR):
            b = sbuf(c)
            pltpu.make_async_copy(bufs.at[b], dst(c), out_sems.at[b]).wait()
        return carry

    lax.fori_loop(0, ITEMS_PER_W, item_body, 0)


_mesh = plsc.VectorSubcoreMesh(core_axis_name="c", subcore_axis_name="s")

_sc_call = functools.partial(
    pl.kernel,
    mesh=_mesh,
    out_type=jax.ShapeDtypeStruct((N_SAMPLES, N_PROTO, N_CLASS), jnp.float32),
    scratch_types=[
        pltpu.VMEM((NBUF, CHUNK, CBLK), jnp.float32),
        pltpu.SemaphoreType.DMA((NBUF,)),
        pltpu.SemaphoreType.DMA((NBUF,)),
    ],
)(_tec_body)


def kernel(contributions):
    return _sc_call(contributions)


# speedup vs baseline: 41.7683x; 1.0178x over previous
import functools

import jax
import jax.numpy as jnp
from jax import lax
from jax.experimental import pallas as pl
from jax.experimental.pallas import tpu as pltpu
from jax.experimental.pallas import tpu_sc as plsc

N_SAMPLES, N_PROTO, N_CLASS = 64, 2048, 256
L = 16
NC, NS = 2, 16
NW = NC * NS
CBLK = 128
N_CBLK = N_CLASS // CBLK
G = CBLK // L
N_ITEMS = N_SAMPLES * N_CBLK
ITEMS_PER_W = N_ITEMS // NW
CHUNK = 128
NCHUNK = N_PROTO // CHUNK
NBUF = 7
P1_AHEAD = 3


def _insert(tops, v):
    t1, t2, t3, t4 = tops
    m1 = jnp.maximum(t1, v)
    v1 = jnp.minimum(t1, v)
    m2 = jnp.maximum(t2, v1)
    v2 = jnp.minimum(t2, v1)
    m3 = jnp.maximum(t3, v2)
    v3 = jnp.minimum(t3, v2)
    m4 = jnp.maximum(t4, v3)
    return (m1, m2, m3, m4)


def _tec_body(x_hbm, out_hbm, bufs, in_sems, out_sems):
    wid = lax.axis_index("s") * NC + lax.axis_index("c")
    neg_inf = jnp.full((L,), -jnp.inf, jnp.float32)

    def item_body(j, carry):
        item = j * NW + wid
        s = item // N_CBLK
        cb = (item % N_CBLK) * CBLK

        def src(c):
            return x_hbm.at[s, pl.ds(c * CHUNK, CHUNK), pl.ds(cb, CBLK)]

        def dst(c):
            return out_hbm.at[s, pl.ds(c * CHUNK, CHUNK), pl.ds(cb, CBLK)]

        for c in range(P1_AHEAD):
            pltpu.async_copy(src(c), bufs.at[c], in_sems.at[c])

        glob = [neg_inf] * (4 * G)
        for c in range(NCHUNK):
            b = c % NBUF
            pltpu.make_async_copy(src(c), bufs.at[b], in_sems.at[b]).wait()
            if c + P1_AHEAD < NCHUNK:
                nb = (c + P1_AHEAD) % NBUF
                pltpu.async_copy(src(c + P1_AHEAD), bufs.at[nb],
                                 in_sems.at[nb])
            buf = bufs.at[b]
            for half in range(2):
                init = tuple(glob[4 * half * 4:4 * half * 4 + 16])

                def sweep_body(i, loc, half=half, buf=buf):
                    loc = list(loc)
                    for k in range(2):
                        p = 2 * i + k
                        for g in range(4):
                            gg = half * 4 + g
                            v = buf[p, pl.ds(L * gg, L)]
                            loc[4 * g:4 * g + 4] = _insert(
                                loc[4 * g:4 * g + 4], v)
                    return tuple(loc)

                fin = lax.fori_loop(0, CHUNK // 2, sweep_body, init)
                glob[4 * half * 4:4 * half * 4 + 16] = list(fin)

        t4s = [glob[4 * g + 3] for g in range(G)]

        def mask_chunk(b):
            buf = bufs.at[b]

            def mask_body(i, carry, buf=buf):
                for k in range(4):
                    p = 4 * i + k
                    for g in range(G):
                        v = buf[p, pl.ds(L * g, L)]
                        buf[p, pl.ds(L * g, L)] = jnp.where(
                            v >= t4s[g], 0.0, v)
                return carry

            lax.fori_loop(0, CHUNK // 4, mask_body, 0)

        RES0 = NCHUNK - NBUF
        NSTR = NCHUNK - NBUF

        def sbuf(c):
            return (RES0 % NBUF + c) % NBUF

        for k in range(NBUF):
            b = (RES0 + k) % NBUF
            mask_chunk(b)
            pltpu.async_copy(bufs.at[b], dst(RES0 + k), out_sems.at[b])
            if k >= 2:
                j = k - 2
                bj = (RES0 + j) % NBUF
                pltpu.make_async_copy(bufs.at[bj], dst(RES0 + j),
                                      out_sems.at[bj]).wait()
                pltpu.async_copy(src(j), bufs.at[bj], in_sems.at[bj])
        for j in range(NBUF - 2, NBUF):
            bj = (RES0 + j) % NBUF
            pltpu.make_async_copy(bufs.at[bj], dst(RES0 + j),
                                  out_sems.at[bj]).wait()
            pltpu.async_copy(src(j), bufs.at[bj], in_sems.at[bj])

        for c in range(NSTR):
            b = sbuf(c)
            pltpu.make_async_copy(src(c), bufs.at[b], in_sems.at[b]).wait()
            mask_chunk(b)
            pltpu.async_copy(bufs.at[b], dst(c), out_sems.at[b])
            nxt = c + 2
            if NBUF <= nxt < NSTR:
                prev = nxt - NBUF
                pltpu.make_async_copy(bufs.at[sbuf(prev)], dst(prev),
                                      out_sems.at[sbuf(prev)]).wait()
                pltpu.async_copy(src(nxt), bufs.at[sbuf(nxt)],
                                 in_sems.at[sbuf(nxt)])

        for c in range(NSTR - NBUF, NSTR):
            b = sbuf(c)
            pltpu.make_async_copy(bufs.at[b], dst(c), out_sems.at[b]).wait()
        return carry

    lax.fori_loop(0, ITEMS_PER_W, item_body, 0)


_mesh = plsc.VectorSubcoreMesh(core_axis_name="c", subcore_axis_name="s")

_sc_call = functools.partial(
    pl.kernel,
    mesh=_mesh,
    out_type=jax.ShapeDtypeStruct((N_SAMPLES, N_PROTO, N_CLASS), jnp.float32),
    scratch_types=[
        pltpu.VMEM((NBUF, CHUNK, CBLK), jnp.float32),
        pltpu.SemaphoreType.DMA((NBUF,)),
        pltpu.SemaphoreType.DMA((NBUF,)),
    ],
)(_tec_body)


def kernel(contributions):
    return _sc_call(contributions)
